# baseline (speedup 1.0000x reference)
import functools

import jax
import jax.numpy as jnp
from jax import lax
from jax.experimental import pallas as pl

_BS = 16384
_VPAD = 1024
_KH = _VPAD // 2


@functools.lru_cache(maxsize=None)
def _build(B, V, D):
    nblk = B // _BS

    def body(idx_ref, tabt_ref, out_ref):
        idx16 = idx_ref[0, 0, :].astype(jnp.int16)
        hi = idx16 >= jnp.int16(_KH)
        lo = jnp.where(hi, idx16 - jnp.int16(_KH), idx16)
        io = lax.broadcasted_iota(jnp.int16, (_KH, _BS), 0)
        oh = jnp.where(io == lo[None, :],
                       jnp.bfloat16(1), jnp.bfloat16(0))
        res_a = jnp.dot(tabt_ref[:, :_KH], oh,
                        preferred_element_type=jnp.float32)
        res_b = jnp.dot(tabt_ref[:, _KH:], oh,
                        preferred_element_type=jnp.float32)
        res = jnp.where(hi[None, :], res_b, res_a)
        out_ref[...] = res.T

    return pl.pallas_call(
        body,
        grid=(nblk,),
        in_specs=[
            pl.BlockSpec((1, 1, _BS), lambda i: (i, 0, 0)),
            pl.BlockSpec((D, _VPAD), lambda i: (0, 0)),
        ],
        out_specs=pl.BlockSpec((_BS, D), lambda i: (i, 0)),
        out_shape=jax.ShapeDtypeStruct((B, D), jnp.float32),
    )


def kernel(visit_order, pos_embed):
    R, S = visit_order.shape
    V, D = pos_embed.shape
    B = R * S
    idx = visit_order.reshape(B // _BS, 1, _BS).astype(jnp.int32)
    tabt = jnp.pad(pos_embed, ((0, _VPAD - V), (0, 0))).astype(jnp.bfloat16).T
    out = _build(B, V, D)(idx, tabt)
    return out.reshape(R, S, D)


# speedup vs baseline: 1.3007x; 1.3007x over previous
import functools

import jax
import jax.numpy as jnp
from jax import lax
from jax.experimental import pallas as pl
from jax.experimental.pallas import tpu as pltpu

_BS = 16384
_VPAD = 1024


@functools.lru_cache(maxsize=None)
def _build(B, V, D):
    nblk = B // _BS

    def body(idx_ref, tabt_ref, out_ref):
        idx16 = idx_ref[0, 0, :].astype(jnp.int16)
        io = lax.broadcasted_iota(jnp.int16, (_VPAD, _BS), 0)
        oh = jnp.where(io == idx16[None, :],
                       jnp.bfloat16(1), jnp.bfloat16(0))
        res = jnp.dot(tabt_ref[...], oh, preferred_element_type=jnp.float32)
        out_ref[...] = res.T

    return pl.pallas_call(
        body,
        grid=(nblk,),
        in_specs=[
            pl.BlockSpec((1, 1, _BS), lambda i: (i, 0, 0)),
            pl.BlockSpec((D, _VPAD), lambda i: (0, 0)),
        ],
        out_specs=pl.BlockSpec((_BS, D), lambda i: (i, 0)),
        out_shape=jax.ShapeDtypeStruct((B, D), jnp.float32),
        compiler_params=pltpu.CompilerParams(vmem_limit_bytes=120 * 1024 * 1024),
    )


def kernel(visit_order, pos_embed):
    R, S = visit_order.shape
    V, D = pos_embed.shape
    B = R * S
    idx = visit_order.reshape(B // _BS, 1, _BS).astype(jnp.int32)
    tabt = jnp.pad(pos_embed, ((0, _VPAD - V), (0, 0))).astype(jnp.bfloat16).T
    out = _build(B, V, D)(idx, tabt)
    return out.reshape(R, S, D)
